# baseline (speedup 1.0000x reference)
import functools

import jax
import jax.numpy as jnp
import numpy as np
from jax import lax
from jax.experimental import pallas as pl
from jax.experimental.pallas import tpu as pltpu
from jax.experimental.pallas import tpu_sc as plsc

NUM_EXPERTS = 8
TOP_K = 2
DIM_MODEL = 768
DIM_EXPERT = 128
S = 2048
EPS = 1e-6

_EN = NUM_EXPERTS * DIM_EXPERT
_LP = 128
_BLK = 512
_NC, _NS = 2, 16
_NW = _NC * _NS
_TOK_W = S // _NW
_WORDS_W = _TOK_W * _LP
_NEG = -3.0e38

_EXPANDER = np.concatenate(
    [np.repeat(np.eye(NUM_EXPERTS, dtype=np.float32), DIM_EXPERT, axis=1),
     np.zeros((_LP - NUM_EXPERTS, _EN), np.float32)], axis=0)


def _route_sc_body(lg_hbm, out_hbm, lg_v, co_v):
    wid = lax.axis_index("s") * _NC + lax.axis_index("c")
    base = wid * _WORDS_W
    pltpu.sync_copy(lg_hbm.at[pl.ds(base, _WORDS_W)], lg_v)
    zero = jnp.zeros((16,), jnp.float32)
    iota = lax.broadcasted_iota(jnp.int32, (16,), 0)

    def _zero_step(k, _):
        co_v[pl.ds(k * 16, 16)] = zero
        return 0

    lax.fori_loop(0, _WORDS_W // 16, _zero_step, 0)

    def _chunk_step(c, _):
        row = iota + c * 16
        ls = [plsc.load_gather(lg_v, [row * _LP + e])
              for e in range(NUM_EXPERTS)]
        m1 = ls[0]
        for e in range(1, NUM_EXPERTS):
            m1 = jnp.maximum(m1, ls[e])
        i1 = jnp.full((16,), float(NUM_EXPERTS), jnp.float32)
        for e in range(NUM_EXPERTS - 1, -1, -1):
            i1 = jnp.where(ls[e] == m1, float(e), i1)
        ms = [jnp.where(i1 == float(e), _NEG, ls[e]) for e in range(NUM_EXPERTS)]
        m2 = ms[0]
        for e in range(1, NUM_EXPERTS):
            m2 = jnp.maximum(m2, ms[e])
        i2 = jnp.full((16,), float(NUM_EXPERTS), jnp.float32)
        for e in range(NUM_EXPERTS - 1, -1, -1):
            i2 = jnp.where(ms[e] == m2, float(e), i2)
        w1 = 1.0 / (1.0 + jnp.exp(m2 - m1))
        w2 = 1.0 - w1
        for e in range(NUM_EXPERTS):
            ce = (jnp.where(i1 == float(e), w1, 0.0)
                  + jnp.where(i2 == float(e), w2, 0.0))
            plsc.store_scatter(co_v, [row * _LP + e], ce)
        return 0

    lax.fori_loop(0, _TOK_W // 16, _chunk_step, 0)
    pltpu.sync_copy(co_v, out_hbm.at[pl.ds(base, _WORDS_W)])


_route_sc = functools.partial(
    pl.kernel,
    out_type=jax.ShapeDtypeStruct((S * _LP,), jnp.float32),
    mesh=plsc.VectorSubcoreMesh(
        core_axis_name="c", subcore_axis_name="s",
        num_cores=_NC, num_subcores=_NS),
    scratch_types=[
        pltpu.VMEM((_WORDS_W,), jnp.float32),
        pltpu.VMEM((_WORDS_W,), jnp.float32),
    ],
    compiler_params=pltpu.CompilerParams(needs_layout_passes=False),
)(_route_sc_body)


def _hidden_body(x_ref, nw_ref, gw_ref, win_ref, wg_ref, h_ref, lg_ref,
                 gwp_s, win_s, wg_s):
    @pl.when(pl.program_id(0) == 0)
    def _cache_weights():
        gwp_s[...] = jnp.concatenate(
            [gw_ref[...],
             jnp.zeros((_LP - NUM_EXPERTS, DIM_MODEL), jnp.float32)], axis=0)
        win_s[...] = win_ref[...].reshape(_EN, DIM_MODEL).astype(jnp.bfloat16)
        wg_s[...] = wg_ref[...].reshape(_EN, DIM_MODEL).astype(jnp.bfloat16)

    x = x_ref[...]
    r = jax.lax.rsqrt(jnp.mean(x * x, axis=1, keepdims=True) + EPS)
    y = x * r * nw_ref[...]
    lg_ref[...] = jax.lax.dot_general(
        y, gwp_s[...], (((1,), (1,)), ((), ())),
        preferred_element_type=jnp.float32)
    yb = y.astype(jnp.bfloat16)
    a = jax.lax.dot_general(yb, win_s[...], (((1,), (1,)), ((), ())),
                            preferred_element_type=jnp.float32)
    g = jax.lax.dot_general(yb, wg_s[...], (((1,), (1,)), ((), ())),
                            preferred_element_type=jnp.float32)
    h = a * (1.0 / (1.0 + jnp.exp(-a))) * g
    h_ref[...] = h.astype(jnp.bfloat16)


def _combine_body(h_ref, co_ref, exp_ref, wout_ref, x_ref, o_ref, wout_s):
    @pl.when(pl.program_id(0) == 0)
    def _cache_weights():
        wout_s[...] = wout_ref[...].astype(jnp.bfloat16)

    coexp = jax.lax.dot_general(
        co_ref[...], exp_ref[...], (((1,), (0,)), ((), ())),
        preferred_element_type=jnp.float32)
    scaled = (h_ref[...].astype(jnp.float32) * coexp).astype(jnp.bfloat16)
    acc = x_ref[...]
    for e in range(NUM_EXPERTS):
        acc = acc + jax.lax.dot_general(
            scaled[:, e * DIM_EXPERT:(e + 1) * DIM_EXPERT],
            wout_s[e], (((1,), (1,)), ((), ())),
            preferred_element_type=jnp.float32)
    o_ref[...] = acc


@jax.jit
def kernel(x, gate_w, W_in, W_gate, W_out, norm_w):
    b, s, m = x.shape
    x2 = x.reshape(s, m)
    nw = norm_w.reshape(1, m)

    h, logits = pl.pallas_call(
        _hidden_body,
        grid=(s // _BLK,),
        in_specs=[
            pl.BlockSpec((_BLK, m), lambda i: (i, 0)),
            pl.BlockSpec((1, m), lambda i: (0, 0)),
            pl.BlockSpec(gate_w.shape, lambda i: (0, 0)),
            pl.BlockSpec(W_in.shape, lambda i: (0, 0, 0)),
            pl.BlockSpec(W_gate.shape, lambda i: (0, 0, 0)),
        ],
        out_specs=[
            pl.BlockSpec((_BLK, _EN), lambda i: (i, 0)),
            pl.BlockSpec((_BLK, _LP), lambda i: (i, 0)),
        ],
        out_shape=[
            jax.ShapeDtypeStruct((s, _EN), jnp.bfloat16),
            jax.ShapeDtypeStruct((s, _LP), jnp.float32),
        ],
        scratch_shapes=[
            pltpu.VMEM((_LP, m), jnp.float32),
            pltpu.VMEM((_EN, m), jnp.bfloat16),
            pltpu.VMEM((_EN, m), jnp.bfloat16),
        ],
    )(x2, nw, gate_w, W_in, W_gate)

    coeff = _route_sc(logits.reshape(s * _LP)).reshape(s, _LP)

    out = pl.pallas_call(
        _combine_body,
        grid=(s // _BLK,),
        in_specs=[
            pl.BlockSpec((_BLK, _EN), lambda i: (i, 0)),
            pl.BlockSpec((_BLK, _LP), lambda i: (i, 0)),
            pl.BlockSpec(_EXPANDER.shape, lambda i: (0, 0)),
            pl.BlockSpec(W_out.shape, lambda i: (0, 0, 0)),
            pl.BlockSpec((_BLK, m), lambda i: (i, 0)),
        ],
        out_specs=pl.BlockSpec((_BLK, m), lambda i: (i, 0)),
        out_shape=jax.ShapeDtypeStruct((s, m), jnp.float32),
        scratch_shapes=[pltpu.VMEM(W_out.shape, jnp.bfloat16)],
    )(h, coeff, jnp.asarray(_EXPANDER), W_out, x2)
    return out.reshape(b, s, m)


# speedup vs baseline: 1.0342x; 1.0342x over previous
import functools

import jax
import jax.numpy as jnp
import numpy as np
from jax import lax
from jax.experimental import pallas as pl
from jax.experimental.pallas import tpu as pltpu
from jax.experimental.pallas import tpu_sc as plsc

NUM_EXPERTS = 8
TOP_K = 2
DIM_MODEL = 768
DIM_EXPERT = 128
S = 2048
EPS = 1e-6

_EN = NUM_EXPERTS * DIM_EXPERT
_LP = 128
_BLK = 512
_NC, _NS = 2, 16
_NW = _NC * _NS
_TOK_W = S // _NW
_WORDS_W = _TOK_W * _LP
_NEG = -3.0e38

_EXPANDER = np.concatenate(
    [np.repeat(np.eye(NUM_EXPERTS, dtype=np.float32), DIM_EXPERT, axis=1),
     np.zeros((_LP - NUM_EXPERTS, _EN), np.float32)], axis=0)


def _route_sc_body(lg_hbm, out_hbm, lg_v, co_v):
    wid = lax.axis_index("s") * _NC + lax.axis_index("c")
    base = wid * _WORDS_W
    pltpu.sync_copy(lg_hbm.at[pl.ds(base, _WORDS_W)], lg_v)
    zero = jnp.zeros((16,), jnp.float32)
    iota = lax.broadcasted_iota(jnp.int32, (16,), 0)

    for k in range(_WORDS_W // 16):
        co_v[pl.ds(k * 16, 16)] = zero
    for c in range(_TOK_W // 16):
        row = iota + c * 16
        ls = [plsc.load_gather(lg_v, [row * _LP + e])
              for e in range(NUM_EXPERTS)]
        m1 = ls[0]
        for e in range(1, NUM_EXPERTS):
            m1 = jnp.maximum(m1, ls[e])
        i1 = jnp.full((16,), float(NUM_EXPERTS), jnp.float32)
        for e in range(NUM_EXPERTS - 1, -1, -1):
            i1 = jnp.where(ls[e] == m1, float(e), i1)
        ms = [jnp.where(i1 == float(e), _NEG, ls[e]) for e in range(NUM_EXPERTS)]
        m2 = ms[0]
        for e in range(1, NUM_EXPERTS):
            m2 = jnp.maximum(m2, ms[e])
        i2 = jnp.full((16,), float(NUM_EXPERTS), jnp.float32)
        for e in range(NUM_EXPERTS - 1, -1, -1):
            i2 = jnp.where(ms[e] == m2, float(e), i2)
        w1 = 1.0 / (1.0 + jnp.exp(m2 - m1))
        w2 = 1.0 - w1
        for e in range(NUM_EXPERTS):
            ce = (jnp.where(i1 == float(e), w1, 0.0)
                  + jnp.where(i2 == float(e), w2, 0.0))
            plsc.store_scatter(co_v, [row * _LP + e], ce)
    pltpu.sync_copy(co_v, out_hbm.at[pl.ds(base, _WORDS_W)])


_route_sc = functools.partial(
    pl.kernel,
    out_type=jax.ShapeDtypeStruct((S * _LP,), jnp.float32),
    mesh=plsc.VectorSubcoreMesh(
        core_axis_name="c", subcore_axis_name="s",
        num_cores=_NC, num_subcores=_NS),
    scratch_types=[
        pltpu.VMEM((_WORDS_W,), jnp.float32),
        pltpu.VMEM((_WORDS_W,), jnp.float32),
    ],
    compiler_params=pltpu.CompilerParams(needs_layout_passes=False),
)(_route_sc_body)


def _hidden_body(x_ref, nw_ref, gw_ref, win_ref, wg_ref, h_ref, lg_ref):
    x = x_ref[...]
    r = jax.lax.rsqrt(jnp.mean(x * x, axis=1, keepdims=True) + EPS)
    y = x * r * nw_ref[...]
    gwp = jnp.concatenate(
        [gw_ref[...], jnp.zeros((_LP - NUM_EXPERTS, DIM_MODEL), jnp.float32)],
        axis=0)
    lg_ref[...] = jax.lax.dot_general(
        y, gwp, (((1,), (1,)), ((), ())),
        preferred_element_type=jnp.float32)
    yb = y.astype(jnp.bfloat16)
    win = win_ref[...].reshape(_EN, DIM_MODEL).astype(jnp.bfloat16)
    wg = wg_ref[...].reshape(_EN, DIM_MODEL).astype(jnp.bfloat16)
    a = jax.lax.dot_general(yb, win, (((1,), (1,)), ((), ())),
                            preferred_element_type=jnp.float32)
    g = jax.lax.dot_general(yb, wg, (((1,), (1,)), ((), ())),
                            preferred_element_type=jnp.float32)
    h = a * (1.0 / (1.0 + jnp.exp(-a))) * g
    h_ref[...] = h.astype(jnp.bfloat16)


def _combine_body(h_ref, co_ref, exp_ref, wout_ref, x_ref, o_ref):
    coexp = jax.lax.dot_general(
        co_ref[...], exp_ref[...], (((1,), (0,)), ((), ())),
        preferred_element_type=jnp.float32)
    scaled = (h_ref[...].astype(jnp.float32) * coexp).astype(jnp.bfloat16)
    acc = x_ref[...]
    for e in range(NUM_EXPERTS):
        acc = acc + jax.lax.dot_general(
            scaled[:, e * DIM_EXPERT:(e + 1) * DIM_EXPERT],
            wout_ref[e].astype(jnp.bfloat16), (((1,), (1,)), ((), ())),
            preferred_element_type=jnp.float32)
    o_ref[...] = acc


@jax.jit
def kernel(x, gate_w, W_in, W_gate, W_out, norm_w):
    b, s, m = x.shape
    x2 = x.reshape(s, m)
    nw = norm_w.reshape(1, m)

    h, logits = pl.pallas_call(
        _hidden_body,
        grid=(s // _BLK,),
        in_specs=[
            pl.BlockSpec((_BLK, m), lambda i: (i, 0)),
            pl.BlockSpec((1, m), lambda i: (0, 0)),
            pl.BlockSpec(gate_w.shape, lambda i: (0, 0)),
            pl.BlockSpec(W_in.shape, lambda i: (0, 0, 0)),
            pl.BlockSpec(W_gate.shape, lambda i: (0, 0, 0)),
        ],
        out_specs=[
            pl.BlockSpec((_BLK, _EN), lambda i: (i, 0)),
            pl.BlockSpec((_BLK, _LP), lambda i: (i, 0)),
        ],
        out_shape=[
            jax.ShapeDtypeStruct((s, _EN), jnp.bfloat16),
            jax.ShapeDtypeStruct((s, _LP), jnp.float32),
        ],
    )(x2, nw, gate_w, W_in, W_gate)

    coeff = _route_sc(logits.reshape(s * _LP)).reshape(s, _LP)

    out = pl.pallas_call(
        _combine_body,
        grid=(s // _BLK,),
        in_specs=[
            pl.BlockSpec((_BLK, _EN), lambda i: (i, 0)),
            pl.BlockSpec((_BLK, _LP), lambda i: (i, 0)),
            pl.BlockSpec(_EXPANDER.shape, lambda i: (0, 0)),
            pl.BlockSpec(W_out.shape, lambda i: (0, 0, 0)),
            pl.BlockSpec((_BLK, m), lambda i: (i, 0)),
        ],
        out_specs=pl.BlockSpec((_BLK, m), lambda i: (i, 0)),
        out_shape=jax.ShapeDtypeStruct((s, m), jnp.float32),
    )(h, coeff, jnp.asarray(_EXPANDER), W_out, x2)
    return out.reshape(b, s, m)
